# Initial kernel scaffold; baseline (speedup 1.0000x reference)
import jax
import jax.numpy as jnp
from jax.experimental import pallas as pl


def kernel(x, edge_index, W, b, gamma, beta):
    raise NotImplementedError("write your pallas kernel here")


# speedup vs baseline: 11.0470x; 11.0470x over previous
import functools

import jax
import jax.numpy as jnp
from jax import lax
from jax.experimental import pallas as pl
from jax.experimental.pallas import tpu as pltpu
import jax.experimental.pallas.tpu_sc as plsc

N = 10000
E = 320000
D = 128

_NS = 16
_STRIPE = 632
_LAST = N - _STRIPE * (_NS - 1)
_EDGES_PER_TILE = E // _NS
_CHUNK = 128
_NFULL = _EDGES_PER_TILE // _CHUNK
_TAIL = _EDGES_PER_TILE - _NFULL * _CHUNK

_sc_mesh = plsc.VectorSubcoreMesh(core_axis_name="c", subcore_axis_name="s")


@functools.partial(
    pl.kernel,
    out_type=jax.ShapeDtypeStruct((N,), jnp.float32),
    mesh=_sc_mesh,
    scratch_types=[
        pltpu.VMEM_SHARED((N,), jnp.float32),
        pltpu.VMEM((_CHUNK,), jnp.int32),
        pltpu.VMEM((_TAIL,), jnp.int32),
        pltpu.VMEM((_CHUNK,), jnp.float32),
    ],
)
def _deg_kernel(dst_hbm, zdeg_hbm, ones_hbm, deg_hbm, deg_sh, dstv, dstv_t, onesv):
    cid = lax.axis_index("c")
    sid = lax.axis_index("s")

    @pl.when(cid == 0)
    def _():
        @pl.when(sid == 0)
        def _():
            pltpu.sync_copy(zdeg_hbm, deg_sh)

        pltpu.sync_copy(ones_hbm, onesv)
        plsc.subcore_barrier()

        base0 = sid * _EDGES_PER_TILE

        def body(i, carry):
            pltpu.sync_copy(dst_hbm.at[pl.ds(base0 + i * _CHUNK, _CHUNK)], dstv)
            pltpu.sync_copy(onesv, deg_sh.at[dstv], add=True)
            return carry

        lax.fori_loop(0, _NFULL, body, 0)
        pltpu.sync_copy(dst_hbm.at[pl.ds(base0 + _NFULL * _CHUNK, _TAIL)], dstv_t)
        pltpu.sync_copy(onesv.at[pl.ds(0, _TAIL)], deg_sh.at[dstv_t], add=True)

        plsc.subcore_barrier()

        @pl.when(sid == 0)
        def _():
            pltpu.sync_copy(deg_sh, deg_hbm)


@functools.partial(
    pl.kernel,
    out_type=[
        jax.ShapeDtypeStruct((N, D), jnp.float32),
        jax.ShapeDtypeStruct((N, D), jnp.float32),
    ],
    mesh=_sc_mesh,
    scratch_types=[
        pltpu.VMEM_SHARED((N, D), jnp.float32),
        pltpu.VMEM((_CHUNK,), jnp.int32),
        pltpu.VMEM((_CHUNK,), jnp.int32),
        pltpu.VMEM((_TAIL,), jnp.int32),
        pltpu.VMEM((_TAIL,), jnp.int32),
        pltpu.VMEM((_CHUNK, D), jnp.float32),
        pltpu.VMEM((_TAIL, D), jnp.float32),
        pltpu.SemaphoreType.DMA,
    ],
)
def _scatter_kernel(tg_hbm, tu_hbm, src_hbm, dst_hbm, zrow_hbm,
                    accg_hbm, accu_hbm,
                    acc_sh, srcv, dstv, srcv_t, dstv_t, rows, rows_t, sem):
    cid = lax.axis_index("c")
    sid = lax.axis_index("s")

    @pl.when(sid < _NS - 1)
    def _():
        pltpu.sync_copy(zrow_hbm, acc_sh.at[pl.ds(sid * _STRIPE, _STRIPE)])

    @pl.when(sid == _NS - 1)
    def _():
        pltpu.sync_copy(zrow_hbm.at[pl.ds(0, _LAST)],
                        acc_sh.at[pl.ds((_NS - 1) * _STRIPE, _LAST)])

    plsc.subcore_barrier()

    base0 = sid * _EDGES_PER_TILE

    def run(table_hbm):
        def chunk(base, sv, dv, rv):
            pltpu.sync_copy(src_hbm.at[pl.ds(base, sv.shape[0])], sv)
            pltpu.sync_copy(dst_hbm.at[pl.ds(base, dv.shape[0])], dv)
            pltpu.async_copy(table_hbm.at[sv], rv, sem).wait()
            pltpu.sync_copy(rv, acc_sh.at[dv], add=True)

        def body(i, carry):
            chunk(base0 + i * _CHUNK, srcv, dstv, rows)
            return carry

        lax.fori_loop(0, _NFULL, body, 0)
        chunk(base0 + _NFULL * _CHUNK, srcv_t, dstv_t, rows_t)

    @pl.when(cid == 0)
    def _():
        run(tg_hbm)

    @pl.when(cid == 1)
    def _():
        run(tu_hbm)

    plsc.subcore_barrier()

    def writeout(out_hbm):
        @pl.when(sid < _NS - 1)
        def _():
            pltpu.sync_copy(acc_sh.at[pl.ds(sid * _STRIPE, _STRIPE)],
                            out_hbm.at[pl.ds(sid * _STRIPE, _STRIPE)])

        @pl.when(sid == _NS - 1)
        def _():
            pltpu.sync_copy(acc_sh.at[pl.ds((_NS - 1) * _STRIPE, _LAST)],
                            out_hbm.at[pl.ds((_NS - 1) * _STRIPE, _LAST)])

    @pl.when(cid == 0)
    def _():
        writeout(accg_hbm)

    @pl.when(cid == 1)
    def _():
        writeout(accu_hbm)


def _prep_body(x_ref, w_ref, b_ref, deg_ref, tg_ref, tu_ref, r_ref):
    h = jnp.dot(x_ref[...], w_ref[...], preferred_element_type=jnp.float32)
    h = h + b_ref[...]
    r = lax.rsqrt(jnp.maximum(deg_ref[...], 1.0))
    nrm2 = jnp.sum(h * h, axis=1, keepdims=True)
    inv_n = lax.rsqrt(jnp.maximum(nrm2, 1e-24))
    tg_ref[...] = h * r
    tu_ref[...] = h * inv_n
    r_ref[...] = r


_prep_call = pl.pallas_call(
    _prep_body,
    out_shape=[
        jax.ShapeDtypeStruct((N, D), jnp.float32),
        jax.ShapeDtypeStruct((N, D), jnp.float32),
        jax.ShapeDtypeStruct((N, 1), jnp.float32),
    ],
)


def _finish_body(accg_ref, accu_ref, tu_ref, r_ref, x_ref, g_ref, be_ref,
                 out_ref, gcd_ref):
    agg = r_ref[...] * accg_ref[...]
    mean = jnp.mean(agg, axis=0, keepdims=True)
    cent = agg - mean
    var = jnp.mean(cent * cent, axis=0, keepdims=True)
    y = cent * lax.rsqrt(var + 1e-5) * g_ref[...] + be_ref[...]
    out_ref[...] = jnp.maximum(y, 0.0) + x_ref[...]
    cos_sum = jnp.sum(tu_ref[...] * accu_ref[...])
    gcd_ref[...] = jnp.reshape(1.0 - cos_sum * (1.0 / E), (1, 1))


_finish_call = pl.pallas_call(
    _finish_body,
    out_shape=[
        jax.ShapeDtypeStruct((N, D), jnp.float32),
        jax.ShapeDtypeStruct((1, 1), jnp.float32),
    ],
)


def kernel(x, edge_index, W, b, gamma, beta):
    src = edge_index[0]
    dst = edge_index[1]
    zdeg = jnp.zeros((N,), jnp.float32)
    ones = jnp.ones((_CHUNK,), jnp.float32)
    zrow = jnp.zeros((_STRIPE, D), jnp.float32)

    deg = _deg_kernel(dst, zdeg, ones)
    tg, tu, r = _prep_call(x, W, b.reshape(1, D), deg.reshape(N, 1))
    accg, accu = _scatter_kernel(tg, tu, src, dst, zrow)
    out, gcd = _finish_call(accg, accu, tu, r, x,
                            gamma.reshape(1, D), beta.reshape(1, D))
    return out, gcd[0, 0]


# speedup vs baseline: 20.4020x; 1.8468x over previous
import functools

import jax
import jax.numpy as jnp
from jax import lax
from jax.experimental import pallas as pl
from jax.experimental.pallas import tpu as pltpu
import jax.experimental.pallas.tpu_sc as plsc

N = 10000
E = 320000
D = 128

_NS = 16
_STRIPE = 632
_LAST = N - _STRIPE * (_NS - 1)
_CHUNK = 128
_CPT = 160
_NCHUNK = _CPT * _NS
_EPAD = _NCHUNK * _CHUNK
_NJUNK = 8
_CPD = _NCHUNK // 32
_G = 32

_sc_mesh = plsc.VectorSubcoreMesh(core_axis_name="c", subcore_axis_name="s")


@functools.partial(
    pl.kernel,
    out_type=[
        jax.ShapeDtypeStruct((N + _NJUNK,), jnp.float32),
        jax.ShapeDtypeStruct((N + _NJUNK,), jnp.float32),
    ],
    mesh=_sc_mesh,
    scratch_types=[
        pltpu.VMEM_SHARED((N + _NJUNK,), jnp.float32),
        pltpu.VMEM((_CPD, _CHUNK), jnp.int32),
        pltpu.VMEM((_CHUNK,), jnp.float32),
        pltpu.SemaphoreType.DMA,
    ],
)
def _deg_kernel(dst2_hbm, zdeg_hbm, ones_hbm, degp0_hbm, degp1_hbm,
                deg_sh, dstv_all, onesv, semd):
    cid = lax.axis_index("c")
    sid = lax.axis_index("s")

    @pl.when(sid == 0)
    def _():
        pltpu.sync_copy(zdeg_hbm, deg_sh)

    pltpu.sync_copy(ones_hbm, onesv)
    rowbase = cid * (_NCHUNK // 2) + sid * _CPD
    pltpu.sync_copy(dst2_hbm.at[pl.ds(rowbase, _CPD)], dstv_all)
    plsc.subcore_barrier()

    def fire(i, carry):
        pltpu.async_copy(onesv, deg_sh.at[dstv_all.at[i]], semd, add=True)
        return carry

    lax.fori_loop(0, _CPD, fire, 0)

    def drain(i, carry):
        pltpu.make_async_copy(onesv, deg_sh.at[dstv_all.at[i]], semd).wait()
        return carry

    lax.fori_loop(0, _CPD, drain, 0)
    plsc.subcore_barrier()

    @pl.when(sid == 0)
    def _():
        @pl.when(cid == 0)
        def _():
            pltpu.sync_copy(deg_sh, degp0_hbm)

        @pl.when(cid == 1)
        def _():
            pltpu.sync_copy(deg_sh, degp1_hbm)


@functools.partial(
    pl.kernel,
    out_type=[
        jax.ShapeDtypeStruct((N, D), jnp.float32),
        jax.ShapeDtypeStruct((N, D), jnp.float32),
    ],
    mesh=_sc_mesh,
    scratch_types=[
        pltpu.VMEM_SHARED((N + _NJUNK, D), jnp.float32),
        pltpu.VMEM((_G, _CHUNK), jnp.int32),
        pltpu.VMEM((_G, _CHUNK), jnp.int32),
        pltpu.VMEM((_CHUNK, D), jnp.float32),
        pltpu.VMEM((_CHUNK, D), jnp.float32),
        pltpu.SemaphoreType.DMA,
        pltpu.SemaphoreType.DMA,
        pltpu.SemaphoreType.DMA,
        pltpu.SemaphoreType.DMA,
    ],
)
def _scatter_kernel(tg_hbm, tu_hbm, src2_hbm, dst2_hbm, zrow_hbm,
                    accg_hbm, accu_hbm,
                    acc_sh, srcv_g, dstv_g, rows0, rows1,
                    semg0, semg1, sems0, sems1):
    cid = lax.axis_index("c")
    sid = lax.axis_index("s")

    def stripe_pieces(fn):
        base = sid * _STRIPE
        for j in range(4):
            fn(base + j * 128, 128)

        @pl.when(sid < _NS - 1)
        def _():
            fn(base + 512, 120)

        @pl.when(sid == _NS - 1)
        def _():
            fn(base + 512, 8)

    stripe_pieces(lambda row0, size: pltpu.sync_copy(
        zrow_hbm.at[pl.ds(0, size)], acc_sh.at[pl.ds(row0, size)]))

    plsc.subcore_barrier()

    def run(table_hbm):
        tilebase = sid * _CPT
        for g in range(_CPT // _G):
            gb = tilebase + g * _G
            pltpu.sync_copy(src2_hbm.at[pl.ds(gb, _G)], srcv_g)
            pltpu.sync_copy(dst2_hbm.at[pl.ds(gb, _G)], dstv_g)
            pltpu.async_copy(table_hbm.at[srcv_g.at[0]], rows0, semg0)

            def pair(p, carry):
                i0 = 2 * p
                i1 = i0 + 1
                pltpu.make_async_copy(table_hbm.at[srcv_g.at[i0]], rows0,
                                      semg0).wait()
                s0 = pltpu.async_copy(rows0, acc_sh.at[dstv_g.at[i0]], sems0,
                                      add=True)

                @pl.when(p > 0)
                def _():
                    pltpu.make_async_copy(rows1, acc_sh.at[dstv_g.at[i1 - 2]],
                                          sems1).wait()

                g1 = pltpu.async_copy(table_hbm.at[srcv_g.at[i1]], rows1,
                                      semg1)
                g1.wait()
                s1 = pltpu.async_copy(rows1, acc_sh.at[dstv_g.at[i1]], sems1,
                                      add=True)
                s0.wait()

                @pl.when(p < _G // 2 - 1)
                def _():
                    pltpu.async_copy(table_hbm.at[srcv_g.at[i0 + 2]], rows0,
                                     semg0)

                return carry

            lax.fori_loop(0, _G // 2, pair, 0)
            pltpu.make_async_copy(rows1, acc_sh.at[dstv_g.at[_G - 1]],
                                  sems1).wait()

    @pl.when(cid == 0)
    def _():
        run(tg_hbm)

    @pl.when(cid == 1)
    def _():
        run(tu_hbm)

    plsc.subcore_barrier()

    def writeout(out_hbm):
        stripe_pieces(lambda row0, size: pltpu.sync_copy(
            acc_sh.at[pl.ds(row0, size)], out_hbm.at[pl.ds(row0, size)]))

    @pl.when(cid == 0)
    def _():
        writeout(accg_hbm)

    @pl.when(cid == 1)
    def _():
        writeout(accu_hbm)


def _prep_body(x_ref, w_ref, b_ref, d0_ref, d1_ref, tg_ref, tu_ref, r_ref):
    h = jnp.dot(x_ref[...], w_ref[...], preferred_element_type=jnp.float32)
    h = h + b_ref[...]
    deg = d0_ref[...] + d1_ref[...]
    r = lax.rsqrt(jnp.maximum(deg, 1.0))
    nrm2 = jnp.sum(h * h, axis=1, keepdims=True)
    inv_n = lax.rsqrt(jnp.maximum(nrm2, 1e-24))
    tg_ref[...] = h * r
    tu_ref[...] = h * inv_n
    r_ref[...] = r


_prep_call = pl.pallas_call(
    _prep_body,
    out_shape=[
        jax.ShapeDtypeStruct((N, D), jnp.float32),
        jax.ShapeDtypeStruct((N, D), jnp.float32),
        jax.ShapeDtypeStruct((N, 1), jnp.float32),
    ],
)


def _finish_body(accg_ref, accu_ref, tu_ref, r_ref, x_ref, g_ref, be_ref,
                 out_ref, gcd_ref):
    agg = r_ref[...] * accg_ref[...]
    mean = jnp.mean(agg, axis=0, keepdims=True)
    cent = agg - mean
    var = jnp.mean(cent * cent, axis=0, keepdims=True)
    y = cent * lax.rsqrt(var + 1e-5) * g_ref[...] + be_ref[...]
    out_ref[...] = jnp.maximum(y, 0.0) + x_ref[...]
    cos_sum = jnp.sum(tu_ref[...] * accu_ref[...])
    gcd_ref[...] = jnp.reshape(1.0 - cos_sum * (1.0 / E), (1, 1))


_finish_call = pl.pallas_call(
    _finish_body,
    out_shape=[
        jax.ShapeDtypeStruct((N, D), jnp.float32),
        jax.ShapeDtypeStruct((1, 1), jnp.float32),
    ],
)


def kernel(x, edge_index, W, b, gamma, beta):
    src = edge_index[0]
    dst = edge_index[1]
    npad = _EPAD - E
    pad_src = (jnp.arange(npad, dtype=jnp.int32) * 131) % N
    pad_dst = N + (jnp.arange(npad, dtype=jnp.int32) % _NJUNK)
    src2 = jnp.concatenate([src, pad_src]).reshape(_NCHUNK, _CHUNK)
    dst2 = jnp.concatenate([dst, pad_dst]).reshape(_NCHUNK, _CHUNK)

    zdeg = jnp.zeros((N + _NJUNK,), jnp.float32)
    ones = jnp.ones((_CHUNK,), jnp.float32)
    zrow = jnp.zeros((_STRIPE, D), jnp.float32)

    degp0, degp1 = _deg_kernel(dst2, zdeg, ones)
    tg, tu, r = _prep_call(x, W, b.reshape(1, D),
                           degp0[:N].reshape(N, 1), degp1[:N].reshape(N, 1))
    accg, accu = _scatter_kernel(tg, tu, src2, dst2, zrow)
    out, gcd = _finish_call(accg, accu, tu, r, x,
                            gamma.reshape(1, D), beta.reshape(1, D))
    return out, gcd[0, 0]
